# Initial kernel scaffold; baseline (speedup 1.0000x reference)
import jax
import jax.numpy as jnp
from jax.experimental import pallas as pl


def kernel(loc_preds, conf_preds, loc_targets, conf_targets):
    raise NotImplementedError("write your pallas kernel here")


# speedup vs baseline: 1.4515x; 1.4515x over previous
import jax
import jax.numpy as jnp
from jax.experimental import pallas as pl


def _phase1_kernel(conf_ref, tgt_ref, locp_ref, loct_ref, ce_ref, locacc_ref):
    b = pl.program_id(0)
    x = conf_ref[0]
    y = tgt_ref[0, 0]
    e = jnp.exp(x)
    s = jnp.sum(e, axis=1)
    cls_iota = jax.lax.broadcasted_iota(jnp.int32, x.shape, 1)
    onehot = (cls_iota == y[:, None]).astype(jnp.float32)
    xy = jnp.sum(x * onehot, axis=1)
    ce = jnp.log(s) - xy
    ce_ref[0, 0] = ce

    pos = (y > 0).astype(jnp.float32)
    d = locp_ref[0] - loct_ref[0]
    ad = jnp.abs(d)
    sl1 = jnp.where(ad < 1.0, 0.5 * d * d, ad - 0.5)
    lsum = jnp.sum(jnp.sum(sl1, axis=1) * pos)

    @pl.when(b == 0)
    def _():
        locacc_ref[...] = jnp.zeros_like(locacc_ref)

    locacc_ref[...] = locacc_ref[...] + jnp.full((1, 1), lsum)


def _phase2_kernel(ce_ref, tgt_ref, locacc_ref, out_ref):
    ce = ce_ref[...]
    y = tgt_ref[...]
    n = ce.shape[1]
    posf = (y > 0).astype(jnp.float32)
    num_pos = jnp.sum(posf, axis=1, keepdims=True)
    num_matched = jnp.sum(num_pos)
    ce_pos_sum = jnp.sum(ce * posf)
    cl = ce * (1.0 - posf)
    bits = jax.lax.bitcast_convert_type(cl, jnp.int32)
    k = jnp.clip(3 * num_pos.astype(jnp.int32), 1, n - 1)

    hi0 = jnp.max(bits, axis=1, keepdims=True)
    lo0 = jnp.zeros_like(hi0)

    def body(_, carry):
        lo, hi = carry
        mid = lo + ((hi - lo + 1) >> 1)
        cnt = jnp.sum((bits >= mid).astype(jnp.int32), axis=1, keepdims=True)
        pred = cnt >= k
        return jnp.where(pred, mid, lo), jnp.where(pred, hi, mid - 1)

    tbits, _ = jax.lax.fori_loop(0, 31, body, (lo0, hi0))
    t = jax.lax.bitcast_convert_type(tbits, jnp.float32)
    gt = bits > tbits
    g = jnp.sum(gt.astype(jnp.float32), axis=1, keepdims=True)
    sgt = jnp.sum(jnp.where(gt, cl, 0.0), axis=1, keepdims=True)
    sneg = sgt + (k.astype(jnp.float32) - g) * t

    total = (locacc_ref[0, 0] + ce_pos_sum + jnp.sum(sneg)) / num_matched
    out_ref[...] = jnp.full((1, 1), total)


def kernel(loc_preds, conf_preds, loc_targets, conf_targets):
    B, N, C = conf_preds.shape
    tgt3 = conf_targets.astype(jnp.int32).reshape(B, 1, N)

    ce3, locacc = pl.pallas_call(
        _phase1_kernel,
        grid=(B,),
        in_specs=[
            pl.BlockSpec((1, N, C), lambda b: (b, 0, 0)),
            pl.BlockSpec((1, 1, N), lambda b: (b, 0, 0)),
            pl.BlockSpec((1, N, 4), lambda b: (b, 0, 0)),
            pl.BlockSpec((1, N, 4), lambda b: (b, 0, 0)),
        ],
        out_specs=[
            pl.BlockSpec((1, 1, N), lambda b: (b, 0, 0)),
            pl.BlockSpec((1, 1), lambda b: (0, 0)),
        ],
        out_shape=[
            jax.ShapeDtypeStruct((B, 1, N), jnp.float32),
            jax.ShapeDtypeStruct((1, 1), jnp.float32),
        ],
    )(conf_preds, tgt3, loc_preds, loc_targets)

    out = pl.pallas_call(
        _phase2_kernel,
        in_specs=[
            pl.BlockSpec((B, N), lambda: (0, 0)),
            pl.BlockSpec((B, N), lambda: (0, 0)),
            pl.BlockSpec((1, 1), lambda: (0, 0)),
        ],
        out_specs=pl.BlockSpec((1, 1), lambda: (0, 0)),
        out_shape=jax.ShapeDtypeStruct((1, 1), jnp.float32),
    )(ce3.reshape(B, N), conf_targets.astype(jnp.int32), locacc)

    return out[0, 0]


# speedup vs baseline: 2.2229x; 1.5314x over previous
import jax
import jax.numpy as jnp
from jax.experimental import pallas as pl


def _dot_rows(w, m):
    return jax.lax.dot_general(w, m, (((1,), (1,)), ((), ())),
                               preferred_element_type=jnp.float32)


def _phase1_kernel(conf_ref, tgt_ref, locp_ref, loct_ref, ce_ref, locacc_ref):
    b = pl.program_id(0)
    x = conf_ref[0]
    y = tgt_ref[0, 0]
    n, c = x.shape
    ones_c = jnp.ones((1, c), jnp.float32)
    e = jnp.exp(x)
    s_row = _dot_rows(ones_c, e)
    cls_iota = jax.lax.broadcasted_iota(jnp.int32, x.shape, 1)
    onehot = (cls_iota == y[:, None]).astype(jnp.float32)
    xy_row = _dot_rows(ones_c, x * onehot)
    ce_ref[0] = jnp.log(s_row) - xy_row

    w_pos = jnp.where(jax.lax.broadcasted_iota(jnp.int32, (1, c), 1) == 0,
                      0.0, 1.0)
    pos_row = _dot_rows(w_pos, onehot)

    d = locp_ref[0] - loct_ref[0]
    ad = jnp.abs(d)
    sl1 = jnp.where(ad < 1.0, 0.5 * d * d, ad - 0.5)
    loc_row = _dot_rows(jnp.ones((1, 4), jnp.float32), sl1)
    lsum = jnp.sum(loc_row * pos_row)

    @pl.when(b == 0)
    def _():
        locacc_ref[...] = jnp.zeros_like(locacc_ref)

    locacc_ref[...] = locacc_ref[...] + jnp.full((1, 1), lsum)


def _phase2_kernel(ce_ref, tgt_ref, locacc_ref, out_ref):
    ce = ce_ref[...]
    y = tgt_ref[...]
    n = ce.shape[1]
    posf = (y > 0).astype(jnp.float32)
    num_pos = jnp.sum(posf, axis=1, keepdims=True)
    num_matched = jnp.sum(num_pos)
    ce_pos_sum = jnp.sum(ce * posf)
    cl = ce * (1.0 - posf)
    bits = jax.lax.bitcast_convert_type(cl, jnp.int32)
    k = jnp.clip(3 * num_pos.astype(jnp.int32), 1, n - 1)

    hi0 = jnp.max(bits, axis=1, keepdims=True)
    lo0 = jnp.zeros_like(hi0)

    def body(_, carry):
        lo, hi = carry
        mid = lo + ((hi - lo + 1) >> 1)
        cnt = jnp.sum((bits >= mid).astype(jnp.int32), axis=1, keepdims=True)
        pred = cnt >= k
        return jnp.where(pred, mid, lo), jnp.where(pred, hi, mid - 1)

    tbits, _ = jax.lax.fori_loop(0, 31, body, (lo0, hi0))
    t = jax.lax.bitcast_convert_type(tbits, jnp.float32)
    gt = bits > tbits
    g = jnp.sum(gt.astype(jnp.float32), axis=1, keepdims=True)
    sgt = jnp.sum(jnp.where(gt, cl, 0.0), axis=1, keepdims=True)
    sneg = sgt + (k.astype(jnp.float32) - g) * t

    total = (locacc_ref[0, 0] + ce_pos_sum + jnp.sum(sneg)) / num_matched
    out_ref[...] = jnp.full((1, 1), total)


def kernel(loc_preds, conf_preds, loc_targets, conf_targets):
    B, N, C = conf_preds.shape
    tgt3 = conf_targets.astype(jnp.int32).reshape(B, 1, N)

    ce3, locacc = pl.pallas_call(
        _phase1_kernel,
        grid=(B,),
        in_specs=[
            pl.BlockSpec((1, N, C), lambda b: (b, 0, 0)),
            pl.BlockSpec((1, 1, N), lambda b: (b, 0, 0)),
            pl.BlockSpec((1, N, 4), lambda b: (b, 0, 0)),
            pl.BlockSpec((1, N, 4), lambda b: (b, 0, 0)),
        ],
        out_specs=[
            pl.BlockSpec((1, 1, N), lambda b: (b, 0, 0)),
            pl.BlockSpec((1, 1), lambda b: (0, 0)),
        ],
        out_shape=[
            jax.ShapeDtypeStruct((B, 1, N), jnp.float32),
            jax.ShapeDtypeStruct((1, 1), jnp.float32),
        ],
    )(conf_preds, tgt3, loc_preds, loc_targets)

    out = pl.pallas_call(
        _phase2_kernel,
        in_specs=[
            pl.BlockSpec((B, N), lambda: (0, 0)),
            pl.BlockSpec((B, N), lambda: (0, 0)),
            pl.BlockSpec((1, 1), lambda: (0, 0)),
        ],
        out_specs=pl.BlockSpec((1, 1), lambda: (0, 0)),
        out_shape=jax.ShapeDtypeStruct((1, 1), jnp.float32),
    )(ce3.reshape(B, N), conf_targets.astype(jnp.int32), locacc)

    return out[0, 0]


# speedup vs baseline: 4.3534x; 1.9584x over previous
import jax
import jax.numpy as jnp
from jax.experimental import pallas as pl

_GROUP = 148
_CROW = 21 * _GROUP
_LROW = 4 * _GROUP


def _dn(c_lhs, c_rhs):
    return (((c_lhs,), (c_rhs,)), ((), ()))


def _phase1_kernel(conf_ref, tgt_ref, locp_ref, loct_ref, w21_ref, w4_ref,
                   ce_ref, locacc_ref):
    b = pl.program_id(0)
    x = conf_ref[0]
    yf = tgt_ref[0]
    w21 = w21_ref[...]
    w4 = w4_ref[...]

    yexp = jax.lax.dot_general(yf, w21, _dn(1, 1),
                               preferred_element_type=jnp.float32)
    cpat = jnp.mod(jax.lax.broadcasted_iota(jnp.int32, (1, _CROW), 1),
                   21).astype(jnp.float32)
    xoh = jnp.where(yexp == cpat, x, 0.0)
    e = jnp.exp(x)
    s = jax.lax.dot_general(e, w21, _dn(1, 0),
                            preferred_element_type=jnp.float32)
    xy = jax.lax.dot_general(xoh, w21, _dn(1, 0),
                             preferred_element_type=jnp.float32)
    ce_ref[0] = jnp.log(s) - xy

    pos = (yf > 0).astype(jnp.float32)
    d = locp_ref[0] - loct_ref[0]
    ad = jnp.abs(d)
    sl1 = jnp.where(ad < 1.0, 0.5 * d * d, ad - 0.5)
    locred = jax.lax.dot_general(sl1, w4, _dn(1, 0),
                                 preferred_element_type=jnp.float32)
    lsum = jnp.sum(locred * pos)

    @pl.when(b == 0)
    def _():
        locacc_ref[...] = jnp.zeros_like(locacc_ref)

    locacc_ref[...] = locacc_ref[...] + jnp.full((1, 1), lsum)


def _phase2_kernel(ce_ref, tgt_ref, locacc_ref, out_ref):
    ce = ce_ref[...]
    yf = tgt_ref[...]
    n = ce.shape[1]
    posf = (yf > 0).astype(jnp.float32)
    num_pos = jnp.sum(posf, axis=1, keepdims=True)
    num_matched = jnp.sum(num_pos)
    ce_pos_sum = jnp.sum(ce * posf)
    cl = ce * (1.0 - posf)
    bits = jax.lax.bitcast_convert_type(cl, jnp.int32)
    k = jnp.clip(3 * num_pos.astype(jnp.int32), 1, n - 1)

    hi0 = jnp.max(bits, axis=1, keepdims=True)
    lo0 = jnp.zeros_like(hi0)

    def body(_, carry):
        lo, hi = carry
        mid = lo + ((hi - lo + 1) >> 1)
        cnt = jnp.sum((bits >= mid).astype(jnp.int32), axis=1, keepdims=True)
        pred = cnt >= k
        return jnp.where(pred, mid, lo), jnp.where(pred, hi, mid - 1)

    tbits, _ = jax.lax.fori_loop(0, 31, body, (lo0, hi0))
    t = jax.lax.bitcast_convert_type(tbits, jnp.float32)
    gt = bits > tbits
    g = jnp.sum(gt.astype(jnp.float32), axis=1, keepdims=True)
    sgt = jnp.sum(jnp.where(gt, cl, 0.0), axis=1, keepdims=True)
    sneg = sgt + (k.astype(jnp.float32) - g) * t

    total = (locacc_ref[0, 0] + ce_pos_sum + jnp.sum(sneg)) / num_matched
    out_ref[...] = jnp.full((1, 1), total)


def kernel(loc_preds, conf_preds, loc_targets, conf_targets):
    B, N, C = conf_preds.shape
    R = N * C // _CROW
    conf2 = conf_preds.reshape(B, R, _CROW)
    locp2 = loc_preds.reshape(B, R, _LROW)
    loct2 = loc_targets.reshape(B, R, _LROW)
    tgtf = conf_targets.astype(jnp.float32).reshape(B, R, _GROUP)
    seg = jnp.arange(_GROUP, dtype=jnp.int32)[None, :]
    w21 = (jnp.arange(_CROW, dtype=jnp.int32)[:, None] // C == seg)
    w21 = w21.astype(jnp.float32)
    w4 = (jnp.arange(_LROW, dtype=jnp.int32)[:, None] // 4 == seg)
    w4 = w4.astype(jnp.float32)

    ce3, locacc = pl.pallas_call(
        _phase1_kernel,
        grid=(B,),
        in_specs=[
            pl.BlockSpec((1, R, _CROW), lambda b: (b, 0, 0)),
            pl.BlockSpec((1, R, _GROUP), lambda b: (b, 0, 0)),
            pl.BlockSpec((1, R, _LROW), lambda b: (b, 0, 0)),
            pl.BlockSpec((1, R, _LROW), lambda b: (b, 0, 0)),
            pl.BlockSpec((_CROW, _GROUP), lambda b: (0, 0)),
            pl.BlockSpec((_LROW, _GROUP), lambda b: (0, 0)),
        ],
        out_specs=[
            pl.BlockSpec((1, R, _GROUP), lambda b: (b, 0, 0)),
            pl.BlockSpec((1, 1), lambda b: (0, 0)),
        ],
        out_shape=[
            jax.ShapeDtypeStruct((B, R, _GROUP), jnp.float32),
            jax.ShapeDtypeStruct((1, 1), jnp.float32),
        ],
    )(conf2, tgtf, locp2, loct2, w21, w4)

    out = pl.pallas_call(
        _phase2_kernel,
        in_specs=[
            pl.BlockSpec((B, N), lambda: (0, 0)),
            pl.BlockSpec((B, N), lambda: (0, 0)),
            pl.BlockSpec((1, 1), lambda: (0, 0)),
        ],
        out_specs=pl.BlockSpec((1, 1), lambda: (0, 0)),
        out_shape=jax.ShapeDtypeStruct((1, 1), jnp.float32),
    )(ce3.reshape(B, N), tgtf.reshape(B, N), locacc)

    return out[0, 0]


# speedup vs baseline: 4.6805x; 1.0751x over previous
import jax
import jax.numpy as jnp
from jax.experimental import pallas as pl

_GROUP = 148
_CROW = 21 * _GROUP
_LROW = 4 * _GROUP


def _dn(c_lhs, c_rhs):
    return (((c_lhs,), (c_rhs,)), ((), ()))


_IB = 4


def _phase1_kernel(conf_ref, tgt_ref, locp_ref, loct_ref, w21_ref, w4_ref,
                   ce_ref, locacc_ref):
    b = pl.program_id(0)
    ib, r, _ = conf_ref.shape
    x = conf_ref[...].reshape(ib * r, _CROW)
    yf = tgt_ref[...].reshape(ib * r, _GROUP)
    w21 = w21_ref[...]
    w4 = w4_ref[...]

    yexp = jax.lax.dot_general(yf, w21, _dn(1, 1),
                               preferred_element_type=jnp.float32)
    cpat = jnp.mod(jax.lax.broadcasted_iota(jnp.int32, (1, _CROW), 1),
                   21).astype(jnp.float32)
    xoh = jnp.where(yexp == cpat, x, 0.0)
    e = jnp.exp(x)
    s = jax.lax.dot_general(e, w21, _dn(1, 0),
                            preferred_element_type=jnp.float32)
    xy = jax.lax.dot_general(xoh, w21, _dn(1, 0),
                             preferred_element_type=jnp.float32)
    ce_ref[...] = (jnp.log(s) - xy).reshape(ib, r, _GROUP)

    pos = (yf > 0).astype(jnp.float32)
    d = (locp_ref[...] - loct_ref[...]).reshape(ib * r, _LROW)
    ad = jnp.abs(d)
    sl1 = jnp.where(ad < 1.0, 0.5 * d * d, ad - 0.5)
    locred = jax.lax.dot_general(sl1, w4, _dn(1, 0),
                                 preferred_element_type=jnp.float32)
    lsum = jnp.sum(locred * pos)

    @pl.when(b == 0)
    def _():
        locacc_ref[...] = jnp.zeros_like(locacc_ref)

    locacc_ref[...] = locacc_ref[...] + jnp.full((1, 1), lsum)


def _phase2_kernel(ce_ref, tgt_ref, locacc_ref, out_ref):
    ce = ce_ref[...]
    yf = tgt_ref[...]
    n = ce.shape[1]
    posf = (yf > 0).astype(jnp.float32)
    num_pos = jnp.sum(posf, axis=1, keepdims=True)
    num_matched = jnp.sum(num_pos)
    ce_pos_sum = jnp.sum(ce * posf)
    cl = ce * (1.0 - posf)
    bits = jax.lax.bitcast_convert_type(cl, jnp.int32)
    k = jnp.clip(3 * num_pos.astype(jnp.int32), 1, n - 1)

    hi0 = jnp.max(bits, axis=1, keepdims=True)
    lo0 = jnp.zeros_like(hi0)

    def body(_, carry):
        lo, hi = carry
        mid = lo + ((hi - lo + 1) >> 1)
        cnt = jnp.sum((bits >= mid).astype(jnp.int32), axis=1, keepdims=True)
        pred = cnt >= k
        return jnp.where(pred, mid, lo), jnp.where(pred, hi, mid - 1)

    tbits, _ = jax.lax.fori_loop(0, 31, body, (lo0, hi0))
    t = jax.lax.bitcast_convert_type(tbits, jnp.float32)
    gt = bits > tbits
    g = jnp.sum(gt.astype(jnp.float32), axis=1, keepdims=True)
    sgt = jnp.sum(jnp.where(gt, cl, 0.0), axis=1, keepdims=True)
    sneg = sgt + (k.astype(jnp.float32) - g) * t

    total = (locacc_ref[0, 0] + ce_pos_sum + jnp.sum(sneg)) / num_matched
    out_ref[...] = jnp.full((1, 1), total)


def kernel(loc_preds, conf_preds, loc_targets, conf_targets):
    B, N, C = conf_preds.shape
    R = N * C // _CROW
    conf2 = conf_preds.reshape(B, R, _CROW)
    locp2 = loc_preds.reshape(B, R, _LROW)
    loct2 = loc_targets.reshape(B, R, _LROW)
    tgtf = conf_targets.astype(jnp.float32).reshape(B, R, _GROUP)
    seg = jnp.arange(_GROUP, dtype=jnp.int32)[None, :]
    w21 = (jnp.arange(_CROW, dtype=jnp.int32)[:, None] // C == seg)
    w21 = w21.astype(jnp.float32)
    w4 = (jnp.arange(_LROW, dtype=jnp.int32)[:, None] // 4 == seg)
    w4 = w4.astype(jnp.float32)

    ce3, locacc = pl.pallas_call(
        _phase1_kernel,
        grid=(B // _IB,),
        in_specs=[
            pl.BlockSpec((_IB, R, _CROW), lambda b: (b, 0, 0)),
            pl.BlockSpec((_IB, R, _GROUP), lambda b: (b, 0, 0)),
            pl.BlockSpec((_IB, R, _LROW), lambda b: (b, 0, 0)),
            pl.BlockSpec((_IB, R, _LROW), lambda b: (b, 0, 0)),
            pl.BlockSpec((_CROW, _GROUP), lambda b: (0, 0)),
            pl.BlockSpec((_LROW, _GROUP), lambda b: (0, 0)),
        ],
        out_specs=[
            pl.BlockSpec((_IB, R, _GROUP), lambda b: (b, 0, 0)),
            pl.BlockSpec((1, 1), lambda b: (0, 0)),
        ],
        out_shape=[
            jax.ShapeDtypeStruct((B, R, _GROUP), jnp.float32),
            jax.ShapeDtypeStruct((1, 1), jnp.float32),
        ],
    )(conf2, tgtf, locp2, loct2, w21, w4)

    out = pl.pallas_call(
        _phase2_kernel,
        in_specs=[
            pl.BlockSpec((B, N), lambda: (0, 0)),
            pl.BlockSpec((B, N), lambda: (0, 0)),
            pl.BlockSpec((1, 1), lambda: (0, 0)),
        ],
        out_specs=pl.BlockSpec((1, 1), lambda: (0, 0)),
        out_shape=jax.ShapeDtypeStruct((1, 1), jnp.float32),
    )(ce3.reshape(B, N), tgtf.reshape(B, N), locacc)

    return out[0, 0]


# speedup vs baseline: 32.1161x; 6.8616x over previous
import jax
import jax.numpy as jnp
from jax.experimental import pallas as pl

_IB = 8


def _phase1_kernel(conf_ref, tgt_ref, locp_ref, loct_ref, ce_ref, locacc_ref):
    b = pl.program_id(0)
    c = conf_ref.shape[0]
    y = tgt_ref[...]
    x0 = conf_ref[0]
    s = jnp.exp(x0)
    xy = jnp.where(y == 0, x0, 0.0)
    for ci in range(1, c):
        xc = conf_ref[ci]
        s = s + jnp.exp(xc)
        xy = jnp.where(y == ci, xc, xy)
    ce_ref[...] = jnp.log(s) - xy

    pos = (y > 0).astype(jnp.float32)
    d = locp_ref[...] - loct_ref[...]
    ad = jnp.abs(d)
    sl1 = jnp.where(ad < 1.0, 0.5 * d * d, ad - 0.5)
    lsum = jnp.sum(jnp.sum(sl1, axis=1) * pos)

    @pl.when(b == 0)
    def _():
        locacc_ref[...] = jnp.zeros_like(locacc_ref)

    locacc_ref[...] = locacc_ref[...] + jnp.full((1, 1), lsum)


def _phase2_kernel(ce_ref, tgt_ref, locacc_ref, out_ref):
    ce = ce_ref[...]
    y = tgt_ref[...]
    n = ce.shape[1]
    posf = (y > 0).astype(jnp.float32)
    num_pos = jnp.sum(posf, axis=1, keepdims=True)
    num_matched = jnp.sum(num_pos)
    ce_pos_sum = jnp.sum(ce * posf)
    cl = ce * (1.0 - posf)
    bits = jax.lax.bitcast_convert_type(cl, jnp.int32)
    k = jnp.clip(3 * num_pos.astype(jnp.int32), 1, n - 1)

    hi0 = jnp.max(bits, axis=1, keepdims=True)
    lo0 = jnp.zeros_like(hi0)

    def body(_, carry):
        lo, hi = carry
        mid = lo + ((hi - lo + 1) >> 1)
        cnt = jnp.sum((bits >= mid).astype(jnp.int32), axis=1, keepdims=True)
        pred = cnt >= k
        return jnp.where(pred, mid, lo), jnp.where(pred, hi, mid - 1)

    tbits, _ = jax.lax.fori_loop(0, 31, body, (lo0, hi0))
    t = jax.lax.bitcast_convert_type(tbits, jnp.float32)
    gt = bits > tbits
    g = jnp.sum(gt.astype(jnp.float32), axis=1, keepdims=True)
    sgt = jnp.sum(jnp.where(gt, cl, 0.0), axis=1, keepdims=True)
    sneg = sgt + (k.astype(jnp.float32) - g) * t

    total = (locacc_ref[0, 0] + ce_pos_sum + jnp.sum(sneg)) / num_matched
    out_ref[...] = jnp.full((1, 1), total)


def kernel(loc_preds, conf_preds, loc_targets, conf_targets):
    B, N, C = conf_preds.shape
    conf_t = jnp.transpose(conf_preds, (2, 0, 1))
    locp_t = jnp.transpose(loc_preds, (0, 2, 1))
    loct_t = jnp.transpose(loc_targets, (0, 2, 1))
    tgt = conf_targets.astype(jnp.int32)

    ce, locacc = pl.pallas_call(
        _phase1_kernel,
        grid=(B // _IB,),
        in_specs=[
            pl.BlockSpec((C, _IB, N), lambda b: (0, b, 0)),
            pl.BlockSpec((_IB, N), lambda b: (b, 0)),
            pl.BlockSpec((_IB, 4, N), lambda b: (b, 0, 0)),
            pl.BlockSpec((_IB, 4, N), lambda b: (b, 0, 0)),
        ],
        out_specs=[
            pl.BlockSpec((_IB, N), lambda b: (b, 0)),
            pl.BlockSpec((1, 1), lambda b: (0, 0)),
        ],
        out_shape=[
            jax.ShapeDtypeStruct((B, N), jnp.float32),
            jax.ShapeDtypeStruct((1, 1), jnp.float32),
        ],
    )(conf_t, tgt, locp_t, loct_t)

    out = pl.pallas_call(
        _phase2_kernel,
        in_specs=[
            pl.BlockSpec((B, N), lambda: (0, 0)),
            pl.BlockSpec((B, N), lambda: (0, 0)),
            pl.BlockSpec((1, 1), lambda: (0, 0)),
        ],
        out_specs=pl.BlockSpec((1, 1), lambda: (0, 0)),
        out_shape=jax.ShapeDtypeStruct((1, 1), jnp.float32),
    )(ce, tgt, locacc)

    return out[0, 0]
